# baseline (speedup 1.0000x reference)
import functools

import jax
import jax.numpy as jnp
from jax import lax
from jax.experimental import pallas as pl
from jax.experimental.pallas import tpu as pltpu
from jax.experimental.pallas import tpu_sc as plsc

ROWS = 4 * 4096
D = 2048
NC = 2
NS = 16
NW = NC * NS
CHUNK = ROWS // NW
_THRESH = 1258291


def _mask16(rows_i32):
    ks0 = jnp.int32(0)
    ks1 = jnp.int32(42)
    ks2 = jnp.int32(0x1BD11BDA ^ 42)
    ks = (ks0, ks1, ks2)
    rot_a = (13, 15, 26, 6)
    rot_b = (17, 29, 16, 24)

    x0 = jnp.zeros_like(rows_i32) + ks0
    x1 = rows_i32 + ks1
    for g in range(5):
        for r in (rot_a if g % 2 == 0 else rot_b):
            x0 = x0 + x1
            x1 = (x1 << r) | lax.shift_right_logical(x1, 32 - r)
            x1 = x1 ^ x0
        x0 = x0 + ks[(g + 1) % 3]
        x1 = x1 + ks[(g + 2) % 3] + jnp.int32(g + 1)
    bits = x0 ^ x1
    shifted = lax.shift_right_logical(bits, 9)
    return jnp.where(shifted <= jnp.int32(_THRESH), jnp.int32(1), jnp.int32(0))


def _sc_body(x_hbm, mv_hbm, out_hbm, mask_v, mv_v, sem):
    wid = lax.axis_index("s") * NC + lax.axis_index("c")
    base = wid * CHUNK

    pltpu.sync_copy(mv_hbm, mv_v)

    def mk(j, carry):
        rows = base + j * 16 + lax.broadcasted_iota(jnp.int32, (16,), 0)
        mask_v[pl.ds(pl.multiple_of(j * 16, 16), 16)] = _mask16(rows)
        return carry

    lax.fori_loop(0, CHUNK // 16, mk, 0)

    def fire(g, carry):
        m16 = mask_v[pl.ds(pl.multiple_of(g * 16, 16), 16)]
        for k in range(16):
            row = base + g * 16 + k
            m = m16[k]

            @pl.when(m == 0)
            def _():
                pltpu.async_copy(x_hbm.at[row], out_hbm.at[row], sem)

            @pl.when(m != 0)
            def _():
                pltpu.async_copy(mv_v, out_hbm.at[row], sem)

        return carry

    lax.fori_loop(0, CHUNK // 16, fire, 0)

    pltpu.make_async_copy(
        x_hbm.at[pl.ds(base, CHUNK)], out_hbm.at[pl.ds(base, CHUNK)], sem
    ).wait()


@jax.jit
def kernel(inputs, mask_value):
    x = inputs.reshape(ROWS, D)
    mesh = plsc.VectorSubcoreMesh(core_axis_name="c", subcore_axis_name="s")
    out = pl.kernel(
        _sc_body,
        out_type=jax.ShapeDtypeStruct((ROWS, D), jnp.float32),
        mesh=mesh,
        scratch_types=[
            pltpu.VMEM((CHUNK,), jnp.int32),
            pltpu.VMEM((D,), jnp.float32),
            pltpu.SemaphoreType.DMA,
        ],
    )(x, mask_value)
    return out.reshape(inputs.shape)


# speedup vs baseline: 41.5053x; 41.5053x over previous
import functools

import jax
import jax.numpy as jnp
from jax import lax
from jax.experimental import pallas as pl

ROWS = 4 * 4096
D = 2048
BLOCK_ROWS = 1024
_THRESH = 1258291


def _threefry_mask(rows_u32):
    ks0 = jnp.uint32(0)
    ks1 = jnp.uint32(42)
    ks2 = jnp.uint32(0x1BD11BDA ^ 42)
    ks = (ks0, ks1, ks2)
    rot_a = (13, 15, 26, 6)
    rot_b = (17, 29, 16, 24)

    x0 = jnp.zeros_like(rows_u32) + ks0
    x1 = rows_u32 + ks1
    for g in range(5):
        for r in (rot_a if g % 2 == 0 else rot_b):
            x0 = x0 + x1
            x1 = (x1 << r) | (x1 >> (32 - r))
            x1 = x1 ^ x0
        x0 = x0 + ks[(g + 1) % 3]
        x1 = x1 + ks[(g + 2) % 3] + jnp.uint32(g + 1)
    bits = x0 ^ x1
    return bits >> 9


def _body(x_ref, mv_ref, o_ref):
    i = pl.program_id(0)
    nsub = BLOCK_ROWS // 128
    rows = jnp.uint32(i * BLOCK_ROWS) + (
        jnp.uint32(128) * lax.broadcasted_iota(jnp.uint32, (nsub, 128), 0)
        + lax.broadcasted_iota(jnp.uint32, (nsub, 128), 1))
    m = jnp.where(_threefry_mask(rows) <= jnp.uint32(_THRESH),
                  jnp.int32(1), jnp.int32(0))
    mt = m.T
    mv = mv_ref[...]
    for j in range(nsub):
        col = mt[:, j:j + 1] != 0
        sl = pl.ds(j * 128, 128)
        o_ref[sl, :] = jnp.where(col, mv, x_ref[sl, :])


@jax.jit
def kernel(inputs, mask_value):
    x = inputs.reshape(ROWS, D)
    mv = mask_value.reshape(1, D)
    out = pl.pallas_call(
        _body,
        grid=(ROWS // BLOCK_ROWS,),
        in_specs=[
            pl.BlockSpec((BLOCK_ROWS, D), lambda i: (i, 0)),
            pl.BlockSpec((1, D), lambda i: (0, 0)),
        ],
        out_specs=pl.BlockSpec((BLOCK_ROWS, D), lambda i: (i, 0)),
        out_shape=jax.ShapeDtypeStruct((ROWS, D), jnp.float32),
    )(x, mv)
    return out.reshape(inputs.shape)
